# baseline (speedup 1.0000x reference)
import functools

import jax
import jax.numpy as jnp
from jax import lax
from jax.experimental import pallas as pl
from jax.experimental.pallas import tpu as pltpu
from jax.experimental.pallas import tpu_sc as plsc

_NC = 2
_NS = 16
_NW = _NC * _NS
_TOTAL = 16 * 64 * 64
_CHUNK = _TOTAL // _NW

_mesh = plsc.VectorSubcoreMesh(core_axis_name="c", subcore_axis_name="s")


@functools.partial(
    pl.kernel,
    mesh=_mesh,
    out_type=jax.ShapeDtypeStruct((_TOTAL,), jnp.float32),
    scratch_types=[
        pltpu.VMEM((16,), jnp.int32),
        pltpu.VMEM((_CHUNK,), jnp.float32),
        pltpu.SemaphoreType.DMA,
    ],
)
def _sc_row_copy(noises_hbm, idx_hbm, out_hbm, idx_v, buf_v, sem):
    wid = lax.axis_index("s") * _NC + lax.axis_index("c")
    base = wid * _CHUNK
    pltpu.sync_copy(idx_hbm, idx_v)
    iv = idx_v[...][0]
    pltpu.async_copy(noises_hbm.at[iv, pl.ds(base, _CHUNK)], buf_v, sem).wait()
    pltpu.sync_copy(buf_v, out_hbm.at[pl.ds(base, _CHUNK)])


def kernel(noises, i):
    flat = noises.reshape(2, _TOTAL)
    idx = jnp.full((16,), i, jnp.int32)
    out = _sc_row_copy(flat, idx)
    return out.reshape(1, 16, 64, 64)
